# baseline (speedup 1.0000x reference)
import functools

import jax
import jax.numpy as jnp
from jax import lax
from jax.experimental import pallas as pl
from jax.experimental.pallas import tpu as pltpu
from jax.experimental.pallas import tpu_sc as plsc

N = 8192
K = 8
N_BLADES = 8
N_FREE = 5
DT = 0.1

ROWS = 256
CHUNK = 128
TOPL = 4
IMAX = 2**31 - 1
IMIN = -2**31

NC = 2
NS = 16
NW = NC * NS


def _knn_kernel(xr_ref, xct_ref, sqr_ref, sqc_ref, o_ref):
    i = pl.program_id(0)
    lane = lax.broadcasted_iota(jnp.int32, (ROWS, CHUNK), 1)
    sqr = sqr_ref[...]
    m = [jnp.full((ROWS, CHUNK), jnp.inf, jnp.float32) for _ in range(TOPL)]
    for c in range(N // CHUNK):
        dot = jnp.dot(xr_ref[...], xct_ref[:, c * CHUNK:(c + 1) * CHUNK],
                      preferred_element_type=jnp.float32)
        d2 = (sqr + sqc_ref[:, c * CHUNK:(c + 1) * CHUNK]) + dot
        bits = lax.bitcast_convert_type(
            jnp.maximum(jnp.sqrt(jnp.maximum(d2, 0.0)), 1e-30), jnp.int32)
        t = lax.bitcast_convert_type((bits & (-64)) | c, jnp.float32)
        for j in range(TOPL):
            nm = jnp.minimum(m[j], t)
            if j < TOPL - 1:
                t = jnp.maximum(m[j], t)
            m[j] = nm
    idxs = []
    for _ in range(K + 1):
        kj = jnp.minimum(jnp.minimum(m[0], m[1]),
                         jnp.minimum(m[2], m[3]))
        kj = jnp.min(kj, axis=1, keepdims=True)
        glane = [jnp.where(v == kj, lane, IMAX) for v in m]
        lj = jnp.minimum(jnp.minimum(glane[0], glane[1]),
                         jnp.minimum(glane[2], glane[3]))
        lj = jnp.min(lj, axis=1, keepdims=True)
        for j in range(TOPL):
            m[j] = jnp.where((glane[j] == lj) & (m[j] == kj), jnp.inf, m[j])
        gj = (lax.bitcast_convert_type(kj, jnp.int32) & 63) * CHUNK + lj
        idxs.append(gj)
    rows = lax.broadcasted_iota(jnp.int32, (ROWS, 1), 0) + i * ROWS
    cum = idxs[0] == rows
    outs = []
    for s in range(K):
        cum = cum | (idxs[s] == rows) if s else cum
        outs.append(jnp.where(cum, idxs[s + 1], idxs[s]))
    o_ref[...] = jnp.concatenate(outs, axis=1)


def _knn(x0):
    sq = jnp.sum(x0 * x0, axis=-1)
    zeros5 = jnp.zeros((N, 5), x0.dtype)
    xr = jnp.concatenate([x0, zeros5], axis=1)
    xct = jnp.concatenate([-2.0 * x0, zeros5], axis=1).T
    return pl.pallas_call(
        _knn_kernel,
        grid=(N // ROWS,),
        in_specs=[
            pl.BlockSpec((ROWS, 8), lambda i: (i, 0)),
            pl.BlockSpec((8, N), lambda i: (0, 0)),
            pl.BlockSpec((ROWS, 1), lambda i: (i, 0)),
            pl.BlockSpec((1, N), lambda i: (0, 0)),
        ],
        out_specs=pl.BlockSpec((ROWS, K), lambda i: (i, 0)),
        out_shape=jax.ShapeDtypeStruct((N, K), jnp.int32),
    )(xr, xct, sq[:, None], sq[None, :])


SC_NODES = N // NS
SC_IDXR = SC_NODES * K // 128


def _sc_gnn_body(h_hbm, col_hbm, w1t_hbm, out_hbm,
                 idx_v, rows_v, own_v, res_v, w1_v, g_v, sem):
    cid = lax.axis_index("c")
    sid = lax.axis_index("s")
    nbase = cid * N + sid * SC_NODES
    pltpu.sync_copy(col_hbm.at[pl.ds((cid * NS + sid) * SC_IDXR, SC_IDXR)],
                    idx_v)
    pltpu.sync_copy(h_hbm.at[pl.ds(nbase, SC_NODES)], own_v)
    pltpu.sync_copy(w1t_hbm, w1_v)
    pltpu.sync_copy(own_v, out_hbm.at[pl.ds(nbase, SC_NODES)])
    plsc.subcore_barrier()

    l16 = lax.iota(jnp.int32, 16)
    perms = [(l16 & (-8)) + i for i in range(N_BLADES)]

    bufs = (own_v, res_v)
    for it in range(N_FREE):
        src, dst = bufs[it % 2], bufs[(it + 1) % 2]
        copies = [
            pltpu.async_copy(out_hbm.at[idx_v.at[j]],
                             rows_v.at[pl.ds(j * 128, 128)], sem)
            for j in range(SC_IDXR)
        ]
        for cp in copies:
            cp.wait()

        def body(n, _):
            acc0 = rows_v[n * K, :] + rows_v[n * K + 1, :]
            acc1 = rows_v[n * K + 2, :] + rows_v[n * K + 3, :]
            acc2 = rows_v[n * K + 4, :] + rows_v[n * K + 5, :]
            acc3 = rows_v[n * K + 6, :] + rows_v[n * K + 7, :]
            own = src[n, :]
            g_v[0, :] = own + (1.0 / K) * ((acc0 + acc1) + (acc2 + acc3))
            za = jnp.zeros((16,), jnp.float32)
            for i in range(N_BLADES):
                gi = plsc.load_gather(g_v.at[0], [perms[i]])
                za = za + gi * w1_v[i, :]
            ez = jnp.exp(2.0 * za)
            th = 1.0 - 2.0 / (ez + 1.0)
            dst[n, :] = own + DT * th
            return 0

        lax.fori_loop(0, SC_NODES, body, 0)
        pltpu.sync_copy(dst, out_hbm.at[pl.ds(nbase, SC_NODES)])
        plsc.subcore_barrier()


@functools.cache
def _sc_gnn():
    mesh = plsc.VectorSubcoreMesh(
        core_axis_name="c", subcore_axis_name="s",
        num_cores=NC, num_subcores=NS)
    return pl.kernel(
        _sc_gnn_body,
        out_type=jax.ShapeDtypeStruct((2 * N, 16), jnp.float32),
        mesh=mesh,
        scratch_types=[
            pltpu.VMEM((SC_IDXR, 128), jnp.int32),
            pltpu.VMEM((SC_NODES * K, 16), jnp.float32),
            pltpu.VMEM((SC_NODES, 16), jnp.float32),
            pltpu.VMEM((SC_NODES, 16), jnp.float32),
            pltpu.VMEM((N_BLADES, 16), jnp.float32),
            pltpu.VMEM((1, 16), jnp.float32),
            pltpu.SemaphoreType.DMA,
        ],
        compiler_params=pltpu.CompilerParams(
            use_tc_tiling_on_sc=False, needs_layout_passes=False),
    )


def _head_kernel(h_ref, w2_ref, fc_ref, b_ref, o_ref):
    pooled = jnp.mean(h_ref[...], axis=0, keepdims=True)
    h2 = pooled
    for _ in range(N_FREE):
        h2 = h2 + DT * jnp.tanh(
            jnp.dot(2.0 * h2, w2_ref[...], preferred_element_type=jnp.float32))
    o_ref[...] = jnp.dot(h2, fc_ref[...],
                         preferred_element_type=jnp.float32) + b_ref[...]


def _head(h, w2blk, fcblk, bias):
    return pl.pallas_call(
        _head_kernel,
        in_specs=[
            pl.BlockSpec((N, 32), lambda: (0, 0)),
            pl.BlockSpec((32, 32), lambda: (0, 0)),
            pl.BlockSpec((32, 16), lambda: (0, 0)),
            pl.BlockSpec((1, 16), lambda: (0, 0)),
        ],
        out_specs=pl.BlockSpec((1, 16), lambda: (0, 0)),
        out_shape=jax.ShapeDtypeStruct((1, 16), jnp.float32),
    )(h, w2blk, fcblk, bias)


def kernel(x, W1, W2, fc_W, fc_b):
    B, n, _ = x.shape
    x0 = x[0]
    nbr = _knn(x0)
    col = nbr.reshape(512, 128)
    col2 = jnp.concatenate([col, col + n], axis=0)

    xt = jnp.transpose(x, (1, 0, 2))
    h0 = jnp.zeros((n, B, N_BLADES), jnp.float32).at[:, :, 1:4].set(xt)
    h2 = jnp.transpose(h0.reshape(n, 2, 2 * N_BLADES), (1, 0, 2))
    h2 = h2.reshape(2 * n, 2 * N_BLADES)

    w1t = jnp.tile(W1, (1, 2))
    hf = _sc_gnn()(h2, col2, w1t)
    h = jnp.transpose(hf.reshape(2, n, 2 * N_BLADES), (1, 0, 2))
    h = h.reshape(n, B * N_BLADES)

    eye = jnp.eye(B, dtype=jnp.float32)
    w2blk = jnp.kron(eye, W2)
    fcblk = jnp.kron(eye, fc_W)
    bias = jnp.tile(fc_b, B)[None, :]
    logits = _head(h, w2blk, fcblk, bias)
    return logits.reshape(B, 4)


# speedup vs baseline: 1.0118x; 1.0118x over previous
import functools

import jax
import jax.numpy as jnp
from jax import lax
from jax.experimental import pallas as pl
from jax.experimental.pallas import tpu as pltpu
from jax.experimental.pallas import tpu_sc as plsc

N = 8192
K = 8
N_BLADES = 8
N_FREE = 5
DT = 0.1

ROWS = 256
CHUNK = 128
TOPL = 4
IMAX = 2**31 - 1
IMIN = -2**31

NC = 2
NS = 16
NW = NC * NS


def _knn_kernel(xr_ref, xct_ref, sqr_ref, sqc_ref, o_ref):
    i = pl.program_id(0)
    lane = lax.broadcasted_iota(jnp.int32, (ROWS, CHUNK), 1)
    sqr = sqr_ref[...]
    m = [jnp.full((ROWS, CHUNK), jnp.inf, jnp.float32) for _ in range(TOPL)]
    for c in range(N // CHUNK):
        dot = jnp.dot(xr_ref[...], xct_ref[:, c * CHUNK:(c + 1) * CHUNK],
                      preferred_element_type=jnp.float32)
        d2 = (sqr + sqc_ref[:, c * CHUNK:(c + 1) * CHUNK]) + dot
        bits = lax.bitcast_convert_type(
            jnp.maximum(jnp.sqrt(jnp.maximum(d2, 0.0)), 1e-30), jnp.int32)
        t = lax.bitcast_convert_type((bits & (-64)) | c, jnp.float32)
        for j in range(TOPL):
            nm = jnp.minimum(m[j], t)
            if j < TOPL - 1:
                t = jnp.maximum(m[j], t)
            m[j] = nm
    idxs = []
    for _ in range(K + 1):
        kj = jnp.minimum(jnp.minimum(m[0], m[1]),
                         jnp.minimum(m[2], m[3]))
        kj = jnp.min(kj, axis=1, keepdims=True)
        glane = [jnp.where(v == kj, lane, IMAX) for v in m]
        lj = jnp.minimum(jnp.minimum(glane[0], glane[1]),
                         jnp.minimum(glane[2], glane[3]))
        lj = jnp.min(lj, axis=1, keepdims=True)
        for j in range(TOPL):
            m[j] = jnp.where((glane[j] == lj) & (m[j] == kj), jnp.inf, m[j])
        gj = (lax.bitcast_convert_type(kj, jnp.int32) & 63) * CHUNK + lj
        idxs.append(gj)
    rows = lax.broadcasted_iota(jnp.int32, (ROWS, 1), 0) + i * ROWS
    cum = idxs[0] == rows
    outs = []
    for s in range(K):
        cum = cum | (idxs[s] == rows) if s else cum
        outs.append(jnp.where(cum, idxs[s + 1], idxs[s]))
    o_ref[...] = jnp.concatenate(outs, axis=1)


def _knn(x0):
    sq = jnp.sum(x0 * x0, axis=-1)
    zeros5 = jnp.zeros((N, 5), x0.dtype)
    xr = jnp.concatenate([x0, zeros5], axis=1)
    xct = jnp.concatenate([-2.0 * x0, zeros5], axis=1).T
    return pl.pallas_call(
        _knn_kernel,
        grid=(N // ROWS,),
        in_specs=[
            pl.BlockSpec((ROWS, 8), lambda i: (i, 0)),
            pl.BlockSpec((8, N), lambda i: (0, 0)),
            pl.BlockSpec((ROWS, 1), lambda i: (i, 0)),
            pl.BlockSpec((1, N), lambda i: (0, 0)),
        ],
        out_specs=pl.BlockSpec((ROWS, K), lambda i: (i, 0)),
        out_shape=jax.ShapeDtypeStruct((N, K), jnp.int32),
    )(xr, xct, sq[:, None], sq[None, :])


SC_NODES = N // NS
SC_IDXR = SC_NODES * K // 128


def _sc_gnn_body(h_hbm, col_hbm, w1t_hbm, out_hbm,
                 idx_v, rows_v, own_v, res_v, w1_v, g_v, sem):
    cid = lax.axis_index("c")
    sid = lax.axis_index("s")
    nbase = cid * N + sid * SC_NODES
    pltpu.sync_copy(col_hbm.at[pl.ds((cid * NS + sid) * SC_IDXR, SC_IDXR)],
                    idx_v)
    pltpu.sync_copy(h_hbm.at[pl.ds(nbase, SC_NODES)], own_v)
    pltpu.sync_copy(w1t_hbm, w1_v)
    pltpu.sync_copy(own_v, out_hbm.at[pl.ds(nbase, SC_NODES)])
    plsc.subcore_barrier()

    l16 = lax.iota(jnp.int32, 16)
    perms = [(l16 & (-8)) + i for i in range(N_BLADES)]
    w1r = [w1_v[i, :] for i in range(N_BLADES)]

    bufs = (own_v, res_v)
    for it in range(N_FREE):
        src, dst = bufs[it % 2], bufs[(it + 1) % 2]
        copies = [
            pltpu.async_copy(out_hbm.at[idx_v.at[j]],
                             rows_v.at[pl.ds(j * 128, 128)], sem)
            for j in range(SC_IDXR)
        ]
        for cp in copies:
            cp.wait()

        def body(nn, _):
            for u in range(2):
                n = nn * 2 + u
                acc0 = rows_v[n * K, :] + rows_v[n * K + 1, :]
                acc1 = rows_v[n * K + 2, :] + rows_v[n * K + 3, :]
                acc2 = rows_v[n * K + 4, :] + rows_v[n * K + 5, :]
                acc3 = rows_v[n * K + 6, :] + rows_v[n * K + 7, :]
                own = src[n, :]
                g_v[u, :] = own + (1.0 / K) * ((acc0 + acc1) + (acc2 + acc3))
                p = [plsc.load_gather(g_v.at[u], [perms[i]]) * w1r[i]
                     for i in range(N_BLADES)]
                za = (((p[0] + p[1]) + (p[2] + p[3])) +
                      ((p[4] + p[5]) + (p[6] + p[7])))
                ez = jnp.exp(2.0 * za)
                th = 1.0 - 2.0 / (ez + 1.0)
                dst[n, :] = own + DT * th
            return 0

        lax.fori_loop(0, SC_NODES // 2, body, 0)
        pltpu.sync_copy(dst, out_hbm.at[pl.ds(nbase, SC_NODES)])
        plsc.subcore_barrier()


@functools.cache
def _sc_gnn():
    mesh = plsc.VectorSubcoreMesh(
        core_axis_name="c", subcore_axis_name="s",
        num_cores=NC, num_subcores=NS)
    return pl.kernel(
        _sc_gnn_body,
        out_type=jax.ShapeDtypeStruct((2 * N, 16), jnp.float32),
        mesh=mesh,
        scratch_types=[
            pltpu.VMEM((SC_IDXR, 128), jnp.int32),
            pltpu.VMEM((SC_NODES * K, 16), jnp.float32),
            pltpu.VMEM((SC_NODES, 16), jnp.float32),
            pltpu.VMEM((SC_NODES, 16), jnp.float32),
            pltpu.VMEM((N_BLADES, 16), jnp.float32),
            pltpu.VMEM((2, 16), jnp.float32),
            pltpu.SemaphoreType.DMA,
        ],
        compiler_params=pltpu.CompilerParams(
            use_tc_tiling_on_sc=False, needs_layout_passes=False),
    )


def _head_kernel(h_ref, w2_ref, fc_ref, b_ref, o_ref):
    pooled = jnp.mean(h_ref[...], axis=0, keepdims=True)
    h2 = pooled
    for _ in range(N_FREE):
        h2 = h2 + DT * jnp.tanh(
            jnp.dot(2.0 * h2, w2_ref[...], preferred_element_type=jnp.float32))
    o_ref[...] = jnp.dot(h2, fc_ref[...],
                         preferred_element_type=jnp.float32) + b_ref[...]


def _head(h, w2blk, fcblk, bias):
    return pl.pallas_call(
        _head_kernel,
        in_specs=[
            pl.BlockSpec((N, 32), lambda: (0, 0)),
            pl.BlockSpec((32, 32), lambda: (0, 0)),
            pl.BlockSpec((32, 16), lambda: (0, 0)),
            pl.BlockSpec((1, 16), lambda: (0, 0)),
        ],
        out_specs=pl.BlockSpec((1, 16), lambda: (0, 0)),
        out_shape=jax.ShapeDtypeStruct((1, 16), jnp.float32),
    )(h, w2blk, fcblk, bias)


def kernel(x, W1, W2, fc_W, fc_b):
    B, n, _ = x.shape
    x0 = x[0]
    nbr = _knn(x0)
    col = nbr.reshape(512, 128)
    col2 = jnp.concatenate([col, col + n], axis=0)

    xt = jnp.transpose(x, (1, 0, 2))
    h0 = jnp.zeros((n, B, N_BLADES), jnp.float32).at[:, :, 1:4].set(xt)
    h2 = jnp.transpose(h0.reshape(n, 2, 2 * N_BLADES), (1, 0, 2))
    h2 = h2.reshape(2 * n, 2 * N_BLADES)

    w1t = jnp.tile(W1, (1, 2))
    hf = _sc_gnn()(h2, col2, w1t)
    h = jnp.transpose(hf.reshape(2, n, 2 * N_BLADES), (1, 0, 2))
    h = h.reshape(n, B * N_BLADES)

    eye = jnp.eye(B, dtype=jnp.float32)
    w2blk = jnp.kron(eye, W2)
    fcblk = jnp.kron(eye, fc_W)
    bias = jnp.tile(fc_b, B)[None, :]
    logits = _head(h, w2blk, fcblk, bias)
    return logits.reshape(B, 4)


# speedup vs baseline: 1.0293x; 1.0174x over previous
import functools

import jax
import jax.numpy as jnp
from jax import lax
from jax.experimental import pallas as pl
from jax.experimental.pallas import tpu as pltpu
from jax.experimental.pallas import tpu_sc as plsc

N = 8192
K = 8
N_BLADES = 8
N_FREE = 5
DT = 0.1

ROWS = 256
CHUNK = 128
TOPL = 4
IMAX = 2**31 - 1
IMIN = -2**31

NC = 2
NS = 16
NW = NC * NS


def _knn_kernel(xr_ref, xct_ref, sqr_ref, sqc_ref, o_ref):
    i = pl.program_id(0)
    lane = lax.broadcasted_iota(jnp.int32, (ROWS, CHUNK), 1)
    sqr = sqr_ref[...]
    m = [jnp.full((ROWS, CHUNK), jnp.inf, jnp.float32) for _ in range(TOPL)]
    for c in range(N // CHUNK):
        dot = jnp.dot(xr_ref[...], xct_ref[:, c * CHUNK:(c + 1) * CHUNK],
                      preferred_element_type=jnp.float32)
        d2 = (sqr + sqc_ref[:, c * CHUNK:(c + 1) * CHUNK]) + dot
        bits = lax.bitcast_convert_type(
            jnp.maximum(jnp.sqrt(jnp.maximum(d2, 0.0)), 1e-30), jnp.int32)
        t = lax.bitcast_convert_type((bits & (-64)) | c, jnp.float32)
        for j in range(TOPL):
            nm = jnp.minimum(m[j], t)
            if j < TOPL - 1:
                t = jnp.maximum(m[j], t)
            m[j] = nm
    idxs = []
    for _ in range(K + 1):
        kj = jnp.minimum(jnp.minimum(m[0], m[1]),
                         jnp.minimum(m[2], m[3]))
        kj = jnp.min(kj, axis=1, keepdims=True)
        glane = [jnp.where(v == kj, lane, IMAX) for v in m]
        lj = jnp.minimum(jnp.minimum(glane[0], glane[1]),
                         jnp.minimum(glane[2], glane[3]))
        lj = jnp.min(lj, axis=1, keepdims=True)
        for j in range(TOPL):
            m[j] = jnp.where((glane[j] == lj) & (m[j] == kj), jnp.inf, m[j])
        gj = (lax.bitcast_convert_type(kj, jnp.int32) & 63) * CHUNK + lj
        idxs.append(gj)
    rows = lax.broadcasted_iota(jnp.int32, (ROWS, 1), 0) + i * ROWS
    cum = idxs[0] == rows
    outs = []
    for s in range(K):
        cum = cum | (idxs[s] == rows) if s else cum
        outs.append(jnp.where(cum, idxs[s + 1], idxs[s]))
    o_ref[...] = jnp.concatenate(outs, axis=1)


def _knn(x0):
    sq = jnp.sum(x0 * x0, axis=-1)
    zeros5 = jnp.zeros((N, 5), x0.dtype)
    xr = jnp.concatenate([x0, zeros5], axis=1)
    xct = jnp.concatenate([-2.0 * x0, zeros5], axis=1).T
    return pl.pallas_call(
        _knn_kernel,
        grid=(N // ROWS,),
        in_specs=[
            pl.BlockSpec((ROWS, 8), lambda i: (i, 0)),
            pl.BlockSpec((8, N), lambda i: (0, 0)),
            pl.BlockSpec((ROWS, 1), lambda i: (i, 0)),
            pl.BlockSpec((1, N), lambda i: (0, 0)),
        ],
        out_specs=pl.BlockSpec((ROWS, K), lambda i: (i, 0)),
        out_shape=jax.ShapeDtypeStruct((N, K), jnp.int32),
    )(xr, xct, sq[:, None], sq[None, :])


SC_NODES = N // NS
SC_IDXR = SC_NODES * K // 128


def _sc_gnn_body(h_hbm, col_hbm, w1t_hbm, out_hbm,
                 idx_v, rows_v, own_v, res_v, w1_v, g_v, sem):
    cid = lax.axis_index("c")
    sid = lax.axis_index("s")
    nbase = cid * N + sid * SC_NODES
    pltpu.sync_copy(col_hbm.at[pl.ds((cid * NS + sid) * SC_IDXR, SC_IDXR)],
                    idx_v)
    pltpu.sync_copy(h_hbm.at[pl.ds(nbase, SC_NODES)], own_v)
    pltpu.sync_copy(w1t_hbm, w1_v)
    pltpu.sync_copy(own_v, out_hbm.at[pl.ds(nbase, SC_NODES)])
    plsc.subcore_barrier()

    l16 = lax.iota(jnp.int32, 16)
    perms = [((l16 & (-8)) + i)[:, None] for i in range(N_BLADES)]
    w1r = [w1_v[i, :] for i in range(N_BLADES)]
    dnums = lax.GatherDimensionNumbers(
        offset_dims=(), collapsed_slice_dims=(0,), start_index_map=(0,))

    def vperm(g, idx):
        return lax.gather(g, idx, dnums, slice_sizes=(1,),
                          mode=lax.GatherScatterMode.PROMISE_IN_BOUNDS)

    bufs = (own_v, res_v)
    for it in range(N_FREE):
        src, dst = bufs[it % 2], bufs[(it + 1) % 2]
        copies = [
            pltpu.async_copy(out_hbm.at[idx_v.at[j]],
                             rows_v.at[pl.ds(j * 128, 128)], sem)
            for j in range(SC_IDXR)
        ]
        for cp in copies:
            cp.wait()

        def body(nn, _):
            for u in range(2):
                n = nn * 2 + u
                acc0 = rows_v[n * K, :] + rows_v[n * K + 1, :]
                acc1 = rows_v[n * K + 2, :] + rows_v[n * K + 3, :]
                acc2 = rows_v[n * K + 4, :] + rows_v[n * K + 5, :]
                acc3 = rows_v[n * K + 6, :] + rows_v[n * K + 7, :]
                own = src[n, :]
                g = own + (1.0 / K) * ((acc0 + acc1) + (acc2 + acc3))
                p = [vperm(g, perms[i]) * w1r[i] for i in range(N_BLADES)]
                za = (((p[0] + p[1]) + (p[2] + p[3])) +
                      ((p[4] + p[5]) + (p[6] + p[7])))
                ez = jnp.exp(2.0 * za)
                th = 1.0 - 2.0 / (ez + 1.0)
                dst[n, :] = own + DT * th
            return 0

        lax.fori_loop(0, SC_NODES // 2, body, 0)
        pltpu.sync_copy(dst, out_hbm.at[pl.ds(nbase, SC_NODES)])
        plsc.subcore_barrier()


@functools.cache
def _sc_gnn():
    mesh = plsc.VectorSubcoreMesh(
        core_axis_name="c", subcore_axis_name="s",
        num_cores=NC, num_subcores=NS)
    return pl.kernel(
        _sc_gnn_body,
        out_type=jax.ShapeDtypeStruct((2 * N, 16), jnp.float32),
        mesh=mesh,
        scratch_types=[
            pltpu.VMEM((SC_IDXR, 128), jnp.int32),
            pltpu.VMEM((SC_NODES * K, 16), jnp.float32),
            pltpu.VMEM((SC_NODES, 16), jnp.float32),
            pltpu.VMEM((SC_NODES, 16), jnp.float32),
            pltpu.VMEM((N_BLADES, 16), jnp.float32),
            pltpu.VMEM((2, 16), jnp.float32),
            pltpu.SemaphoreType.DMA,
        ],
        compiler_params=pltpu.CompilerParams(
            use_tc_tiling_on_sc=False, needs_layout_passes=False),
    )


def _head_kernel(h_ref, w2_ref, fc_ref, b_ref, o_ref):
    pooled = jnp.mean(h_ref[...], axis=0, keepdims=True)
    h2 = pooled
    for _ in range(N_FREE):
        h2 = h2 + DT * jnp.tanh(
            jnp.dot(2.0 * h2, w2_ref[...], preferred_element_type=jnp.float32))
    o_ref[...] = jnp.dot(h2, fc_ref[...],
                         preferred_element_type=jnp.float32) + b_ref[...]


def _head(h, w2blk, fcblk, bias):
    return pl.pallas_call(
        _head_kernel,
        in_specs=[
            pl.BlockSpec((N, 32), lambda: (0, 0)),
            pl.BlockSpec((32, 32), lambda: (0, 0)),
            pl.BlockSpec((32, 16), lambda: (0, 0)),
            pl.BlockSpec((1, 16), lambda: (0, 0)),
        ],
        out_specs=pl.BlockSpec((1, 16), lambda: (0, 0)),
        out_shape=jax.ShapeDtypeStruct((1, 16), jnp.float32),
    )(h, w2blk, fcblk, bias)


def kernel(x, W1, W2, fc_W, fc_b):
    B, n, _ = x.shape
    x0 = x[0]
    nbr = _knn(x0)
    col = nbr.reshape(512, 128)
    col2 = jnp.concatenate([col, col + n], axis=0)

    xt = jnp.transpose(x, (1, 0, 2))
    h0 = jnp.zeros((n, B, N_BLADES), jnp.float32).at[:, :, 1:4].set(xt)
    h2 = jnp.transpose(h0.reshape(n, 2, 2 * N_BLADES), (1, 0, 2))
    h2 = h2.reshape(2 * n, 2 * N_BLADES)

    w1t = jnp.tile(W1, (1, 2))
    hf = _sc_gnn()(h2, col2, w1t)
    h = jnp.transpose(hf.reshape(2, n, 2 * N_BLADES), (1, 0, 2))
    h = h.reshape(n, B * N_BLADES)

    eye = jnp.eye(B, dtype=jnp.float32)
    w2blk = jnp.kron(eye, W2)
    fcblk = jnp.kron(eye, fc_W)
    bias = jnp.tile(fc_b, B)[None, :]
    logits = _head(h, w2blk, fcblk, bias)
    return logits.reshape(B, 4)


# speedup vs baseline: 1.0838x; 1.0529x over previous
import functools

import jax
import jax.numpy as jnp
from jax import lax
from jax.experimental import pallas as pl
from jax.experimental.pallas import tpu as pltpu
from jax.experimental.pallas import tpu_sc as plsc

N = 8192
K = 8
N_BLADES = 8
N_FREE = 5
DT = 0.1

ROWS = 256
CHUNK = 128
TOPL = 4
IMAX = 2**31 - 1
IMIN = -2**31

NC = 2
NS = 16
NW = NC * NS
NODES_W = N // NW
EDGES_W = NODES_W * K
IDX_ROWS = EDGES_W // 128


def _knn_kernel(xr_ref, xct_ref, sqr_ref, sqc_ref, o_ref):
    i = pl.program_id(0)
    lane = lax.broadcasted_iota(jnp.int32, (ROWS, CHUNK), 1)
    sqr = sqr_ref[...]
    m = [jnp.full((ROWS, CHUNK), jnp.inf, jnp.float32) for _ in range(TOPL)]
    for c in range(N // CHUNK):
        dot = jnp.dot(xr_ref[...], xct_ref[:, c * CHUNK:(c + 1) * CHUNK],
                      preferred_element_type=jnp.float32)
        d2 = (sqr + sqc_ref[:, c * CHUNK:(c + 1) * CHUNK]) + dot
        bits = lax.bitcast_convert_type(
            jnp.maximum(jnp.sqrt(jnp.maximum(d2, 0.0)), 1e-30), jnp.int32)
        t = lax.bitcast_convert_type((bits & (-64)) | c, jnp.float32)
        for j in range(TOPL):
            nm = jnp.minimum(m[j], t)
            if j < TOPL - 1:
                t = jnp.maximum(m[j], t)
            m[j] = nm
    idxs = []
    for _ in range(K + 1):
        kj = jnp.minimum(jnp.minimum(m[0], m[1]),
                         jnp.minimum(m[2], m[3]))
        kj = jnp.min(kj, axis=1, keepdims=True)
        glane = [jnp.where(v == kj, lane, IMAX) for v in m]
        lj = jnp.minimum(jnp.minimum(glane[0], glane[1]),
                         jnp.minimum(glane[2], glane[3]))
        lj = jnp.min(lj, axis=1, keepdims=True)
        for j in range(TOPL):
            m[j] = jnp.where((glane[j] == lj) & (m[j] == kj), jnp.inf, m[j])
        gj = (lax.bitcast_convert_type(kj, jnp.int32) & 63) * CHUNK + lj
        idxs.append(gj)
    rows = lax.broadcasted_iota(jnp.int32, (ROWS, 1), 0) + i * ROWS
    cum = idxs[0] == rows
    outs = []
    for s in range(K):
        cum = cum | (idxs[s] == rows) if s else cum
        outs.append(jnp.where(cum, idxs[s + 1], idxs[s]))
    o_ref[...] = jnp.concatenate(outs, axis=1)


def _knn(x0):
    sq = jnp.sum(x0 * x0, axis=-1)
    zeros5 = jnp.zeros((N, 5), x0.dtype)
    xr = jnp.concatenate([x0, zeros5], axis=1)
    xct = jnp.concatenate([-2.0 * x0, zeros5], axis=1).T
    return pl.pallas_call(
        _knn_kernel,
        grid=(N // ROWS,),
        in_specs=[
            pl.BlockSpec((ROWS, 8), lambda i: (i, 0)),
            pl.BlockSpec((8, N), lambda i: (0, 0)),
            pl.BlockSpec((ROWS, 1), lambda i: (i, 0)),
            pl.BlockSpec((1, N), lambda i: (0, 0)),
        ],
        out_specs=pl.BlockSpec((ROWS, K), lambda i: (i, 0)),
        out_shape=jax.ShapeDtypeStruct((N, K), jnp.int32),
    )(xr, xct, sq[:, None], sq[None, :])


def _sc_gather_body(h_hbm, col_hbm, out_hbm, idx_v, rows_v, own_v, res_v, sem):
    wid = lax.axis_index("s") * NC + lax.axis_index("c")
    nbase = wid * NODES_W
    pltpu.sync_copy(col_hbm.at[pl.ds(wid * IDX_ROWS, IDX_ROWS)], idx_v)
    copies = [
        pltpu.async_copy(h_hbm.at[idx_v.at[j]],
                         rows_v.at[pl.ds(j * 128, 128)], sem)
        for j in range(IDX_ROWS)
    ]
    pltpu.sync_copy(h_hbm.at[pl.ds(nbase, NODES_W)], own_v)
    for cp in copies:
        cp.wait()

    def body(nn, _):
        for u in range(4):
            n = nn * 4 + u
            for half in (0, 16):
                acc0 = rows_v[n * K, pl.ds(half, 16)] + \
                    rows_v[n * K + 1, pl.ds(half, 16)]
                acc1 = rows_v[n * K + 2, pl.ds(half, 16)] + \
                    rows_v[n * K + 3, pl.ds(half, 16)]
                acc2 = rows_v[n * K + 4, pl.ds(half, 16)] + \
                    rows_v[n * K + 5, pl.ds(half, 16)]
                acc3 = rows_v[n * K + 6, pl.ds(half, 16)] + \
                    rows_v[n * K + 7, pl.ds(half, 16)]
                acc = (acc0 + acc1) + (acc2 + acc3)
                res_v[n, pl.ds(half, 16)] = (
                    own_v[n, pl.ds(half, 16)] + (1.0 / K) * acc)
        return 0

    lax.fori_loop(0, NODES_W // 4, body, 0)
    pltpu.sync_copy(res_v, out_hbm.at[pl.ds(nbase, NODES_W)])


@functools.cache
def _sc_gather():
    mesh = plsc.VectorSubcoreMesh(
        core_axis_name="c", subcore_axis_name="s",
        num_cores=NC, num_subcores=NS)
    return pl.kernel(
        _sc_gather_body,
        out_type=jax.ShapeDtypeStruct((N, 32), jnp.float32),
        mesh=mesh,
        scratch_types=[
            pltpu.VMEM((IDX_ROWS, 128), jnp.int32),
            pltpu.VMEM((EDGES_W, 32), jnp.float32),
            pltpu.VMEM((NODES_W, 32), jnp.float32),
            pltpu.VMEM((NODES_W, 32), jnp.float32),
            pltpu.SemaphoreType.DMA,
        ],
        compiler_params=pltpu.CompilerParams(use_tc_tiling_on_sc=False),
    )


UROWS = 2048


def _update_kernel(g_ref, w_ref, h_ref, o_ref):
    o_ref[...] = h_ref[...] + DT * jnp.tanh(
        jnp.dot(g_ref[...], w_ref[...], preferred_element_type=jnp.float32))


def _tc_update(g, wblk, h):
    return pl.pallas_call(
        _update_kernel,
        grid=(N // UROWS,),
        in_specs=[
            pl.BlockSpec((UROWS, 32), lambda i: (i, 0)),
            pl.BlockSpec((32, 32), lambda i: (0, 0)),
            pl.BlockSpec((UROWS, 32), lambda i: (i, 0)),
        ],
        out_specs=pl.BlockSpec((UROWS, 32), lambda i: (i, 0)),
        out_shape=jax.ShapeDtypeStruct((N, 32), jnp.float32),
    )(g, wblk, h)


def _head_kernel(h_ref, w2_ref, fc_ref, b_ref, o_ref):
    pooled = jnp.mean(h_ref[...], axis=0, keepdims=True)
    h2 = pooled
    for _ in range(N_FREE):
        h2 = h2 + DT * jnp.tanh(
            jnp.dot(2.0 * h2, w2_ref[...], preferred_element_type=jnp.float32))
    o_ref[...] = jnp.dot(h2, fc_ref[...],
                         preferred_element_type=jnp.float32) + b_ref[...]


def _head(h, w2blk, fcblk, bias):
    return pl.pallas_call(
        _head_kernel,
        in_specs=[
            pl.BlockSpec((N, 32), lambda: (0, 0)),
            pl.BlockSpec((32, 32), lambda: (0, 0)),
            pl.BlockSpec((32, 16), lambda: (0, 0)),
            pl.BlockSpec((1, 16), lambda: (0, 0)),
        ],
        out_specs=pl.BlockSpec((1, 16), lambda: (0, 0)),
        out_shape=jax.ShapeDtypeStruct((1, 16), jnp.float32),
    )(h, w2blk, fcblk, bias)


def kernel(x, W1, W2, fc_W, fc_b):
    B, n, _ = x.shape
    x0 = x[0]
    nbr = _knn(x0)
    col = nbr.reshape(NW * IDX_ROWS, 128)

    xt = jnp.transpose(x, (1, 0, 2))
    h0 = jnp.zeros((n, B, N_BLADES), jnp.float32).at[:, :, 1:4].set(xt)
    h = h0.reshape(n, B * N_BLADES)

    eye = jnp.eye(B, dtype=jnp.float32)
    w1blk = jnp.kron(eye, W1)
    w2blk = jnp.kron(eye, W2)
    fcblk = jnp.kron(eye, fc_W)
    bias = jnp.tile(fc_b, B)[None, :]

    gather = _sc_gather()
    for _ in range(N_FREE):
        g = gather(h, col)
        h = _tc_update(g, w1blk, h)

    logits = _head(h, w2blk, fcblk, bias)
    return logits.reshape(B, 4)
